# Initial kernel scaffold; baseline (speedup 1.0000x reference)
import jax
import jax.numpy as jnp
from jax.experimental import pallas as pl


def kernel(x, dijk, idx_j, seg_i, seg_j, W1, b1, W2, b2, Win, Wout, bout, Wd, bd):
    raise NotImplementedError("write your pallas kernel here")


# speedup vs baseline: 1.8794x; 1.8794x over previous
import functools

import jax
import jax.numpy as jnp
from jax import lax
from jax.experimental import pallas as pl
from jax.experimental.pallas import tpu as pltpu
from jax.experimental.pallas import tpu_sc as plsc

N_ATOMS = 10000
N_INT = 320000
D = 128
F = 128

NC = 2
NS = 16
HALF = N_ATOMS // NC
CONV_R = HALF + 8
EPT = N_INT // NS
K = 80
NCH = EPT // K
ZERO_STEP, ZERO_SLAB = 312, 328
OUT_STEP, OUT_SLAB = 312, 320

_LOG2 = 0.6931471805599453


def _ssp(v):
    m = jnp.maximum(v, 0.0)
    return m + jnp.log(1.0 + jnp.exp(v - 2.0 * m)) - _LOG2


_BE = 2560


def _filter_body(d_ref, w1_ref, b1_ref, w2_ref, b2_ref, o_ref):
    h = _ssp(jnp.dot(d_ref[...], w1_ref[...],
                     preferred_element_type=jnp.float32) + b1_ref[...])
    o_ref[...] = _ssp(jnp.dot(h, w2_ref[...],
                              preferred_element_type=jnp.float32) + b2_ref[...])


_filter = pl.pallas_call(
    _filter_body,
    grid=(N_INT // _BE,),
    in_specs=[
        pl.BlockSpec((_BE, D), lambda i: (i, 0)),
        pl.BlockSpec((D, F), lambda i: (0, 0)),
        pl.BlockSpec((1, F), lambda i: (0, 0)),
        pl.BlockSpec((F, F), lambda i: (0, 0)),
        pl.BlockSpec((1, F), lambda i: (0, 0)),
    ],
    out_specs=pl.BlockSpec((_BE, F), lambda i: (i, 0)),
    out_shape=jax.ShapeDtypeStruct((N_INT, F), jnp.float32),
)


def _in2fac_body(x_ref, win_ref, o_ref):
    o_ref[...] = jnp.dot(x_ref[...], win_ref[...],
                         preferred_element_type=jnp.float32)


_in2fac = pl.pallas_call(
    _in2fac_body,
    out_shape=jax.ShapeDtypeStruct((N_ATOMS, F), jnp.float32),
)


_BA = 2000


def _out_body(x_ref, conv_ref, wout_ref, bout_ref, wd_ref, bd_ref,
              y_ref, v_ref):
    hconv = _ssp(jnp.dot(conv_ref[...], wout_ref[...],
                         preferred_element_type=jnp.float32) + bout_ref[...])
    v = jnp.dot(hconv, wd_ref[...],
                preferred_element_type=jnp.float32) + bd_ref[...]
    v_ref[...] = v
    y_ref[...] = x_ref[...] + v


_outk = pl.pallas_call(
    _out_body,
    grid=(N_ATOMS // _BA,),
    in_specs=[
        pl.BlockSpec((_BA, D), lambda i: (i, 0)),
        pl.BlockSpec((_BA, F), lambda i: (i, 0)),
        pl.BlockSpec((F, D), lambda i: (0, 0)),
        pl.BlockSpec((1, D), lambda i: (0, 0)),
        pl.BlockSpec((D, D), lambda i: (0, 0)),
        pl.BlockSpec((1, D), lambda i: (0, 0)),
    ],
    out_specs=[
        pl.BlockSpec((_BA, D), lambda i: (i, 0)),
        pl.BlockSpec((_BA, D), lambda i: (i, 0)),
    ],
    out_shape=[
        jax.ShapeDtypeStruct((N_ATOMS, D), jnp.float32),
        jax.ShapeDtypeStruct((N_ATOMS, D), jnp.float32),
    ],
)


_sc_mesh = plsc.VectorSubcoreMesh(core_axis_name="c", subcore_axis_name="s")


@functools.partial(
    pl.kernel,
    out_type=jax.ShapeDtypeStruct((NC, HALF, F), jnp.float32),
    mesh=_sc_mesh,
    scratch_types=[
        pltpu.VMEM((NCH, K), jnp.int32),
        pltpu.VMEM((NCH, K), jnp.int32),
        pltpu.VMEM((K, F), jnp.float32),
        pltpu.VMEM((K, F), jnp.float32),
        pltpu.VMEM_SHARED((CONV_R, F), jnp.float32),
        pltpu.SemaphoreType.DMA,
    ],
)
def _sc_conv(w_hbm, f_hbm, idx_hbm, seg_hbm, zeros_hbm, out_hbm,
             idx_v, seg_v, rows_v, wv, conv_sh, sem):
    cid = lax.axis_index("c")
    sid = lax.axis_index("s")

    pltpu.sync_copy(zeros_hbm.at[pl.ds(sid * ZERO_STEP, ZERO_SLAB)],
                    conv_sh.at[pl.ds(sid * ZERO_STEP, ZERO_SLAB)])
    pltpu.sync_copy(idx_hbm.at[sid], idx_v)
    pltpu.sync_copy(seg_hbm.at[cid, sid], seg_v)
    plsc.subcore_barrier()

    def chunk_body(j, carry):
        base = sid * EPT + j * K
        pltpu.sync_copy(w_hbm.at[pl.ds(base, K)], wv)
        pltpu.async_copy(f_hbm.at[idx_v.at[j]], rows_v, sem).wait()

        def mul_row(e, c2):
            for c in range(F // 16):
                s = pl.ds(c * 16, 16)
                rows_v[e, s] = rows_v[e, s] * wv[e, s]
            return c2

        lax.fori_loop(0, K, mul_row, 0)
        pltpu.sync_copy(rows_v, conv_sh.at[seg_v.at[j]], add=True)
        return carry

    lax.fori_loop(0, NCH, chunk_body, 0)
    plsc.subcore_barrier()

    pltpu.sync_copy(conv_sh.at[pl.ds(sid * OUT_STEP, OUT_SLAB)],
                    out_hbm.at[cid, pl.ds(sid * OUT_STEP, OUT_SLAB)])


def kernel(x, dijk, idx_j, seg_i, seg_j, W1, b1, W2, b2, Win, Wout, bout,
           Wd, bd):
    del seg_j
    idx3 = idx_j.astype(jnp.int32).reshape(NS, NCH, K)
    seg32 = seg_i.astype(jnp.int32)
    seg_lo = jnp.where(seg32 < HALF, seg32, HALF)
    seg_hi = jnp.where(seg32 >= HALF, seg32 - HALF, HALF)
    seg4 = jnp.stack([seg_lo, seg_hi]).reshape(NC, NS, NCH, K)
    zeros = jnp.zeros((CONV_R, F), jnp.float32)

    w = _filter(dijk, W1, b1.reshape(1, F), W2, b2.reshape(1, F))
    f = _in2fac(x, Win)
    conv = _sc_conv(w, f, idx3, seg4, zeros).reshape(N_ATOMS, F)
    y, v = _outk(x, conv, Wout, bout.reshape(1, D), Wd, bd.reshape(1, D))
    return (y, v)


# speedup vs baseline: 3.1567x; 1.6796x over previous
import functools

import jax
import jax.numpy as jnp
from jax import lax
from jax.experimental import pallas as pl
from jax.experimental.pallas import tpu as pltpu
from jax.experimental.pallas import tpu_sc as plsc

N_ATOMS = 10000
N_INT = 320000
D = 128
F = 128

NSPL = 2
NE_SPLITS = (160000,) * NSPL
NE_OFF = (0, 160000)

NC = 2
NS = 16
HALF = N_ATOMS // NC
CONV_R = HALF + 8
K = 80
NBUF = 3
ZERO_STEP, ZERO_SLAB = 312, 328
OUT_STEP, OUT_SLAB = 312, 320

_LOG2 = 0.6931471805599453


def _ssp(v):
    m = jnp.maximum(v, 0.0)
    return m + jnp.log(1.0 + jnp.exp(v - 2.0 * m)) - _LOG2


_BE = 2000


def _filter_body(d_ref, w1_ref, b1_ref, w2_ref, b2_ref, o_ref):
    h = _ssp(jnp.dot(d_ref[...], w1_ref[...],
                     preferred_element_type=jnp.float32) + b1_ref[...])
    o_ref[...] = _ssp(jnp.dot(h, w2_ref[...],
                              preferred_element_type=jnp.float32) + b2_ref[...])


def _make_filter(split):
    ne = NE_SPLITS[split]
    off = NE_OFF[split] // _BE
    return pl.pallas_call(
        _filter_body,
        grid=(ne // _BE,),
        in_specs=[
            pl.BlockSpec((_BE, D), lambda i: (i + off, 0)),
            pl.BlockSpec((D, F), lambda i: (0, 0)),
            pl.BlockSpec((1, F), lambda i: (0, 0)),
            pl.BlockSpec((F, F), lambda i: (0, 0)),
            pl.BlockSpec((1, F), lambda i: (0, 0)),
        ],
        out_specs=pl.BlockSpec((_BE, F), lambda i: (i, 0)),
        out_shape=jax.ShapeDtypeStruct((ne, F), jnp.float32),
    )


_filters = [_make_filter(s) for s in range(NSPL)]


def _in2fac_body(x_ref, win_ref, o_ref):
    o_ref[...] = jnp.dot(x_ref[...], win_ref[...],
                         preferred_element_type=jnp.float32)


_in2fac = pl.pallas_call(
    _in2fac_body,
    out_shape=jax.ShapeDtypeStruct((N_ATOMS, F), jnp.float32),
)


_BA = 2000


def _out_body(x_ref, c0_ref, c1_ref, wout_ref, bout_ref, wd_ref, bd_ref,
              y_ref, v_ref):
    conv = c0_ref[...] + c1_ref[...]
    hconv = _ssp(jnp.dot(conv, wout_ref[...],
                         preferred_element_type=jnp.float32) + bout_ref[...])
    v = jnp.dot(hconv, wd_ref[...],
                preferred_element_type=jnp.float32) + bd_ref[...]
    v_ref[...] = v
    y_ref[...] = x_ref[...] + v


_outk = pl.pallas_call(
    _out_body,
    grid=(N_ATOMS // _BA,),
    in_specs=[
        pl.BlockSpec((_BA, D), lambda i: (i, 0)),
        pl.BlockSpec((_BA, F), lambda i: (i, 0)),
        pl.BlockSpec((_BA, F), lambda i: (i, 0)),
        pl.BlockSpec((F, D), lambda i: (0, 0)),
        pl.BlockSpec((1, D), lambda i: (0, 0)),
        pl.BlockSpec((D, D), lambda i: (0, 0)),
        pl.BlockSpec((1, D), lambda i: (0, 0)),
    ],
    out_specs=[
        pl.BlockSpec((_BA, D), lambda i: (i, 0)),
        pl.BlockSpec((_BA, D), lambda i: (i, 0)),
    ],
    out_shape=[
        jax.ShapeDtypeStruct((N_ATOMS, D), jnp.float32),
        jax.ShapeDtypeStruct((N_ATOMS, D), jnp.float32),
    ],
)


_sc_mesh = plsc.VectorSubcoreMesh(core_axis_name="c", subcore_axis_name="s")


def _make_sc(ne):
    ept = ne // NS
    nch = ept // K

    @functools.partial(
        pl.kernel,
        out_type=jax.ShapeDtypeStruct((NC, HALF, F), jnp.float32),
        mesh=_sc_mesh,
        scratch_types=[
            pltpu.VMEM((ept,), jnp.int32),
            pltpu.VMEM((nch, K), jnp.int32),
            [pltpu.VMEM((K, F), jnp.float32) for _ in range(NBUF)],
            [pltpu.VMEM((K, F), jnp.float32) for _ in range(NBUF)],
            pltpu.VMEM_SHARED((CONV_R, F), jnp.float32),
            [pltpu.SemaphoreType.DMA for _ in range(NBUF)],
            [pltpu.SemaphoreType.DMA for _ in range(NBUF)],
            [pltpu.SemaphoreType.DMA for _ in range(NBUF)],
        ],
    )
    def _sc_conv(w_hbm, f_hbm, idx_hbm, seg_hbm, zeros_hbm, out_hbm,
                 idx_v, seg_v, rows_b, wv_b, conv_sh,
                 sem_g, sem_w, sem_s):
        cid = lax.axis_index("c")
        sid = lax.axis_index("s")

        pltpu.sync_copy(zeros_hbm.at[pl.ds(sid * ZERO_STEP, ZERO_SLAB)],
                        conv_sh.at[pl.ds(sid * ZERO_STEP, ZERO_SLAB)])
        pltpu.sync_copy(idx_hbm.at[sid], idx_v)
        pltpu.sync_copy(seg_hbm.at[cid * NS + sid], seg_v)
        plsc.subcore_barrier()

        def issue_pref(b, c):
            base = sid * ept + c * K
            pltpu.async_copy(w_hbm.at[pl.ds(base, K)], wv_b[b], sem_w[b])
            pltpu.async_copy(f_hbm.at[idx_v.at[pl.ds(c * K, K)]], rows_b[b],
                             sem_g[b])

        def step(b, c):
            base = sid * ept + c * K
            b2 = (b + 2) % NBUF
            is_static = isinstance(c, int)

            pltpu.make_async_copy(w_hbm.at[pl.ds(base, K)], wv_b[b],
                                  sem_w[b]).wait()
            pltpu.make_async_copy(f_hbm.at[idx_v.at[pl.ds(c * K, K)]],
                                  rows_b[b], sem_g[b]).wait()

            def mul_row(e, c2):
                for r in range(2):
                    for cc in range(F // 16):
                        s = pl.ds(cc * 16, 16)
                        e2 = 2 * e + r
                        rows_b[b][e2, s] = rows_b[b][e2, s] * wv_b[b][e2, s]
                return c2

            lax.fori_loop(0, K // 2, mul_row, 0)
            pltpu.async_copy(rows_b[b], conv_sh.at[seg_v.at[c]], sem_s[b],
                             add=True)

            prev_c = max(c - 1, 0) if is_static else jnp.maximum(c - 1, 0)

            @pl.when(c >= 1)
            def _wait_prev_scatter():
                pltpu.make_async_copy(rows_b[b2],
                                      conv_sh.at[seg_v.at[prev_c]],
                                      sem_s[b2]).wait()

            if not (is_static and c + 2 > nch - 1):
                nxt_c = c + 2 if is_static else jnp.minimum(c + 2, nch - 1)

                @pl.when(c + 2 <= nch - 1)
                def _prefetch():
                    issue_pref(b2, nxt_c)

        issue_pref(0, 0)
        issue_pref(1, 1)

        def tri_body(i, carry):
            for b in range(NBUF):
                step(b, i * NBUF + b)
            return carry

        full = nch // NBUF
        lax.fori_loop(0, full, tri_body, 0)
        for c in range(full * NBUF, nch):
            step(c % NBUF, c)
        pltpu.make_async_copy(rows_b[(nch - 1) % NBUF],
                              conv_sh.at[seg_v.at[nch - 1]],
                              sem_s[(nch - 1) % NBUF]).wait()
        plsc.subcore_barrier()

        pltpu.sync_copy(conv_sh.at[pl.ds(sid * OUT_STEP, OUT_SLAB)],
                        out_hbm.at[cid, pl.ds(sid * OUT_STEP, OUT_SLAB)])

    return _sc_conv


_sc_convs = [_make_sc(n) for n in NE_SPLITS]


def kernel(x, dijk, idx_j, seg_i, seg_j, W1, b1, W2, b2, Win, Wout, bout,
           Wd, bd):
    del seg_j
    idx32 = idx_j.astype(jnp.int32)
    seg32 = seg_i.astype(jnp.int32)
    seg_lo = jnp.where(seg32 < HALF, seg32, HALF)
    seg_hi = jnp.where(seg32 >= HALF, seg32 - HALF, HALF)
    zeros = jnp.zeros((CONV_R, F), jnp.float32)
    b1r, b2r = b1.reshape(1, F), b2.reshape(1, F)

    f = _in2fac(x, Win)
    convs = []
    for s in range(NSPL):
        ne = NE_SPLITS[s]
        sl = slice(NE_OFF[s], NE_OFF[s] + ne)
        idx2 = idx32[sl].reshape(NS, ne // NS)
        seg2 = jnp.stack([seg_lo[sl], seg_hi[sl]]).reshape(
            NC * NS, ne // NS // K, K)
        w = _filters[s](dijk, W1, b1r, W2, b2r)
        convs.append(_sc_convs[s](w, f, idx2, seg2, zeros)
                     .reshape(N_ATOMS, F))
    y, v = _outk(x, convs[0], convs[1], Wout, bout.reshape(1, D),
                 Wd, bd.reshape(1, D))
    return (y, v)
